# Initial kernel scaffold; baseline (speedup 1.0000x reference)
import jax
import jax.numpy as jnp
from jax.experimental import pallas as pl


def kernel(features, edge_index, edge_weight, train_set, W_emb, b_emb, W_gc1, b_gc1):
    raise NotImplementedError("write your pallas kernel here")


# speedup vs baseline: 1.0394x; 1.0394x over previous
import functools

import jax
import jax.numpy as jnp
from jax import lax
from jax.experimental import pallas as pl
from jax.experimental.pallas import tpu as pltpu

N = 10000
E = 160000
F = 256
D = 256
B = 16384

_ROWS = 1000


def _embed_body(f_ref, w1_ref, b1_ref, w2_ref, o_ref):
    il = jnp.maximum(
        jnp.dot(f_ref[...], w1_ref[...], preferred_element_type=jnp.float32)
        + b1_ref[...],
        0.0,
    )
    o_ref[...] = jnp.maximum(
        jnp.dot(il, w2_ref[...], preferred_element_type=jnp.float32), 0.0
    )


def _embed(features, W_emb, b_emb, W_gc1):
    grid = (N // _ROWS,)
    return pl.pallas_call(
        _embed_body,
        grid=grid,
        in_specs=[
            pl.BlockSpec((_ROWS, F), lambda i: (i, 0)),
            pl.BlockSpec((F, D), lambda i: (0, 0)),
            pl.BlockSpec((1, D), lambda i: (0, 0)),
            pl.BlockSpec((D, D), lambda i: (0, 0)),
        ],
        out_specs=pl.BlockSpec((_ROWS, D), lambda i: (i, 0)),
        out_shape=jax.ShapeDtypeStruct((N, D), jnp.float32),
    )(features, W_emb, b_emb.reshape(1, D), W_gc1)


def _spmm(edge_index, edge_weight, x, n):
    row = edge_index[0]
    col = edge_index[1]
    return jax.ops.segment_sum(edge_weight[:, None] * x[col], row, num_segments=n)


def kernel(features, edge_index, edge_weight, train_set, W_emb, b_emb, W_gc1, b_gc1):
    n = features.shape[0]
    support = _embed(features, W_emb, b_emb, W_gc1)
    As = _spmm(edge_index, edge_weight, support, n)
    output_low = As + support
    AAs = _spmm(edge_index, edge_weight, As, n)
    output_mid = AAs - support
    output_high = support - As
    o1 = output_high * jax.nn.relu(output_low + output_mid)
    o2 = output_mid * jax.nn.relu(output_low + output_high)
    o3 = output_low * jax.nn.relu(output_high + output_mid)
    item_latent = jax.nn.relu(o1 + o2 + o3) + b_gc1
    key_emb = item_latent[train_set[:, 0]]
    pos_emb = item_latent[train_set[:, 1]]
    neg_emb = item_latent[train_set[:, 2]]
    pos_scores = jnp.sum(key_emb * pos_emb, axis=1)
    neg_scores = jnp.sum(key_emb * neg_emb, axis=1)
    loss = -jnp.mean(jnp.log(jax.nn.sigmoid(pos_scores - neg_scores) + 1e-09))
    win = pos_scores >= neg_scores
    mrr = jnp.mean(jnp.where(win, 1e-09, 1.0).astype(jnp.float32))
    hr = jnp.mean(win.astype(jnp.float32))
    ndcg = jnp.mean(jnp.where(win, 1.0, 2.0 / 3.0).astype(jnp.float32))
    return (loss, mrr, hr, ndcg)


# speedup vs baseline: 2.3784x; 2.2882x over previous
import functools

import jax
import jax.numpy as jnp
from jax import lax
from jax.experimental import pallas as pl
from jax.experimental.pallas import tpu as pltpu
from jax.experimental.pallas import tpu_sc as plsc

N = 10000
NPAD = 10240
E = 160000
F = 256
D = 256
B = 16384

NC = 2
NS = 16
L = 16
NQ = 4
QW = D // NQ
NPASS = NQ // NC
CH = 128
NCHUNK = 80
EPAD = NS * NCHUNK * CH
RSTRIPE = NPAD // NS
TSUB = B // NS
TCHUNK = TSUB // CH

_ROWS = 640


def _embed_body(f_ref, w1_ref, b1_ref, w2_ref, o_ref):
    il = jnp.maximum(
        jnp.dot(f_ref[...], w1_ref[...], preferred_element_type=jnp.float32)
        + b1_ref[...],
        0.0,
    )
    s = jnp.maximum(
        jnp.dot(il, w2_ref[...], preferred_element_type=jnp.float32), 0.0
    )
    for q in range(NQ):
        o_ref[q] = s[:, q * QW:(q + 1) * QW]


def _embed(features, W_emb, b_emb, W_gc1):
    return pl.pallas_call(
        _embed_body,
        grid=(NPAD // _ROWS,),
        in_specs=[
            pl.BlockSpec((_ROWS, F), lambda i: (i, 0)),
            pl.BlockSpec((F, D), lambda i: (0, 0)),
            pl.BlockSpec((1, D), lambda i: (0, 0)),
            pl.BlockSpec((D, D), lambda i: (0, 0)),
        ],
        out_specs=pl.BlockSpec((NQ, _ROWS, QW), lambda i: (0, i, 0)),
        out_shape=jax.ShapeDtypeStruct((NQ, NPAD, QW), jnp.float32),
    )(features, W_emb, b_emb.reshape(1, D), W_gc1)


def _sc_body(sup_hbm, rows_hbm, cols_hbm, w_hbm, ki_hbm, pi_hbm, ni_hbm,
             bg_hbm, ps_hbm, ns_hbm, as_hbm,
             shared, rbuf, cbuf, wbuf, big0, big1, big2,
             kibuf, pibuf, nibuf, bgbuf, psvec, nsvec,
             sem0, sem1, semk, semp, semn):
    c = lax.axis_index("c")
    s = lax.axis_index("s")
    base = s * RSTRIPE

    pltpu.sync_copy(rows_hbm.at[s], rbuf)
    pltpu.sync_copy(cols_hbm.at[s], cbuf)
    pltpu.sync_copy(w_hbm.at[s], wbuf)
    pltpu.sync_copy(ki_hbm.at[s], kibuf)
    pltpu.sync_copy(pi_hbm.at[s], pibuf)
    pltpu.sync_copy(ni_hbm.at[s], nibuf)

    def _zero_big0(r, _):
        for k in range(QW // L):
            big0[r, pl.ds(k * L, L)] = jnp.zeros((L,), jnp.float32)
        return 0

    def _zero_stripe():
        lax.fori_loop(0, CH, _zero_big0, 0)
        for t in range(RSTRIPE // CH):
            pltpu.sync_copy(big0, shared.at[pl.ds(base + t * CH, CH)])

    def _scale(buf, j):
        def body(g, _):
            w16 = wbuf[j, pl.ds(g * L, L)]
            for i in range(L):
                e = g * L + i
                w = w16[i]
                for k in range(QW // L):
                    sl = pl.ds(k * L, L)
                    buf[e, sl] = buf[e, sl] * w
            return 0
        lax.fori_loop(0, CH // L, body, 0)

    def _spmm(table):
        pltpu.async_copy(table.at[cbuf.at[0]], big0, sem0)
        pltpu.async_copy(table.at[cbuf.at[1]], big1, sem1)

        def pair(jj, _):
            for b, (buf, sem) in enumerate(((big0, sem0), (big1, sem1))):
                j = jj + b
                pltpu.make_async_copy(table.at[cbuf.at[j]], buf, sem).wait()
                _scale(buf, j)
                pltpu.sync_copy(buf, shared.at[rbuf.at[j]], add=True)

                @pl.when(j + 2 < NCHUNK)
                def _():
                    pltpu.async_copy(table.at[cbuf.at[j + 2]], buf, sem)
            return 0

        lax.fori_loop(0, NCHUNK // 2, lambda i, x: pair(2 * i, x), 0)

    for p in range(NPASS):
        qq = c * NPASS + p
        my_sup = sup_hbm.at[qq]
        my_as = as_hbm.at[qq]
        pltpu.sync_copy(bg_hbm.at[qq], bgbuf)

        _zero_stripe()
        plsc.subcore_barrier()

        _spmm(my_sup)
        plsc.subcore_barrier()

        pltpu.sync_copy(shared.at[pl.ds(base, RSTRIPE)],
                        my_as.at[pl.ds(base, RSTRIPE)])
        _zero_stripe()
        plsc.subcore_barrier()

        _spmm(my_as)
        plsc.subcore_barrier()

        for t in range(RSTRIPE // CH):
            rb = base + t * CH
            pltpu.sync_copy(my_as.at[pl.ds(rb, CH)], big0)
            pltpu.sync_copy(my_sup.at[pl.ds(rb, CH)], big1)
            pltpu.sync_copy(shared.at[pl.ds(rb, CH)], big2)

            def gate(r, _):
                for k in range(QW // L):
                    sl = pl.ds(k * L, L)
                    a_v = big0[r, sl]
                    s_v = big1[r, sl]
                    q_v = big2[r, sl]
                    zero = jnp.zeros((L,), jnp.float32)
                    o1 = (s_v - a_v) * jnp.maximum(a_v + q_v, zero)
                    o2 = (q_v - s_v) * (s_v + s_v)
                    o3 = (a_v + s_v) * jnp.maximum(q_v - a_v, zero)
                    big2[r, sl] = (
                        jnp.maximum(o1 + o2 + o3, zero) + bgbuf[0, sl]
                    )
                return 0

            lax.fori_loop(0, CH, gate, 0)
            pltpu.sync_copy(big2, my_as.at[pl.ds(rb, CH)])
        plsc.subcore_barrier()

        for t in range(TCHUNK):
            ck = pltpu.async_copy(my_as.at[kibuf.at[t]], big0, semk)
            cp = pltpu.async_copy(my_as.at[pibuf.at[t]], big1, semp)
            cn = pltpu.async_copy(my_as.at[nibuf.at[t]], big2, semn)
            ck.wait()
            cp.wait()
            cn.wait()

            def dot(e, _):
                accp = jnp.zeros((L,), jnp.float32)
                accn = jnp.zeros((L,), jnp.float32)
                for k in range(QW // L):
                    sl = pl.ds(k * L, L)
                    kv = big0[e, sl]
                    accp = accp + kv * big1[e, sl]
                    accn = accn + kv * big2[e, sl]
                psvec[0, pl.ds(e * L, L)] = accp
                nsvec[0, pl.ds(e * L, L)] = accn
                return 0

            lax.fori_loop(0, CH, dot, 0)
            pltpu.sync_copy(psvec, ps_hbm.at[c, p, s, t])
            pltpu.sync_copy(nsvec, ns_hbm.at[c, p, s, t])

        plsc.subcore_barrier()


def _sc_call(sup, rows3, cols3, w3, ki3, pi3, ni3, bg3):
    fn = pl.kernel(
        _sc_body,
        out_type=(
            jax.ShapeDtypeStruct((NC, NPASS, NS, TCHUNK, 1, CH * L),
                                 jnp.float32),
            jax.ShapeDtypeStruct((NC, NPASS, NS, TCHUNK, 1, CH * L),
                                 jnp.float32),
            jax.ShapeDtypeStruct((NQ, NPAD, QW), jnp.float32),
        ),
        mesh=plsc.VectorSubcoreMesh(
            core_axis_name="c", subcore_axis_name="s"
        ),
        compiler_params=pltpu.CompilerParams(use_tc_tiling_on_sc=False),
        scratch_types=[
            pltpu.VMEM_SHARED((NPAD, QW), jnp.float32),
            pltpu.VMEM((NCHUNK, CH), jnp.int32),
            pltpu.VMEM((NCHUNK, CH), jnp.int32),
            pltpu.VMEM((NCHUNK, CH), jnp.float32),
            pltpu.VMEM((CH, QW), jnp.float32),
            pltpu.VMEM((CH, QW), jnp.float32),
            pltpu.VMEM((CH, QW), jnp.float32),
            pltpu.VMEM((TCHUNK, CH), jnp.int32),
            pltpu.VMEM((TCHUNK, CH), jnp.int32),
            pltpu.VMEM((TCHUNK, CH), jnp.int32),
            pltpu.VMEM((1, QW), jnp.float32),
            pltpu.VMEM((1, CH * L), jnp.float32),
            pltpu.VMEM((1, CH * L), jnp.float32),
            pltpu.SemaphoreType.DMA,
            pltpu.SemaphoreType.DMA,
            pltpu.SemaphoreType.DMA,
            pltpu.SemaphoreType.DMA,
            pltpu.SemaphoreType.DMA,
        ],
    )
    return fn(sup, rows3, cols3, w3, ki3, pi3, ni3, bg3)


def _finish_body(ps_ref, ns_ref, loss_ref, mrr_ref, hr_ref, ndcg_ref):
    rows = lax.broadcasted_iota(jnp.int32, (128, 8), 0)
    cols = lax.broadcasted_iota(jnp.int32, (128, 8), 1)
    G = jnp.where(rows // L == cols, 1.0, 0.0).astype(jnp.float32)
    pv = ps_ref[0] + ps_ref[1] + ps_ref[2] + ps_ref[3]
    nv = ns_ref[0] + ns_ref[1] + ns_ref[2] + ns_ref[3]
    ps = jnp.dot(pv, G, preferred_element_type=jnp.float32)
    ns = jnp.dot(nv, G, preferred_element_type=jnp.float32)
    d = ps - ns
    loss_ref[...] = (-jnp.mean(jnp.log(jax.nn.sigmoid(d) + 1e-09))).reshape(1, 1)
    win = ps >= ns
    mrr_ref[...] = jnp.mean(jnp.where(win, 1e-09, 1.0)).reshape(1, 1)
    hr_ref[...] = jnp.mean(jnp.where(win, 1.0, 0.0)).reshape(1, 1)
    ndcg_ref[...] = jnp.mean(jnp.where(win, 1.0, 2.0 / 3.0)).reshape(1, 1)


def _finish(ps_parts, ns_parts):
    outs = pl.pallas_call(
        _finish_body,
        in_specs=[
            pl.BlockSpec((4, 2048, 128), lambda: (0, 0, 0)),
            pl.BlockSpec((4, 2048, 128), lambda: (0, 0, 0)),
        ],
        out_specs=[pl.BlockSpec((1, 1), lambda: (0, 0))] * 4,
        out_shape=[jax.ShapeDtypeStruct((1, 1), jnp.float32)] * 4,
    )(ps_parts, ns_parts)
    return tuple(o.reshape(()) for o in outs)


def kernel(features, edge_index, edge_weight, train_set, W_emb, b_emb, W_gc1, b_gc1):
    fpad = jnp.pad(features, ((0, NPAD - N), (0, 0)))
    support = _embed(fpad, W_emb, b_emb, W_gc1)

    pad = EPAD - E
    rows3 = jnp.pad(edge_index[0], (0, pad)).reshape(NS, NCHUNK, CH)
    cols3 = jnp.pad(edge_index[1], (0, pad)).reshape(NS, NCHUNK, CH)
    w3 = jnp.pad(edge_weight, (0, pad)).reshape(NS, NCHUNK, CH)
    ki3 = train_set[:, 0].reshape(NS, TCHUNK, CH)
    pi3 = train_set[:, 1].reshape(NS, TCHUNK, CH)
    ni3 = train_set[:, 2].reshape(NS, TCHUNK, CH)
    bg3 = b_gc1.reshape(NQ, 1, QW)

    ps_parts, ns_parts, _ = _sc_call(
        support, rows3, cols3, w3, ki3, pi3, ni3, bg3
    )
    return _finish(
        ps_parts.reshape(4, 2048, 128), ns_parts.reshape(4, 2048, 128)
    )
